# Initial kernel scaffold; baseline (speedup 1.0000x reference)
import jax
import jax.numpy as jnp
from jax.experimental import pallas as pl


def kernel(edge_index, batch, sphex, W_g2g, b_g2g, W_intra, b_intra):
    raise NotImplementedError("write your pallas kernel here")


# speedup vs baseline: 14.6323x; 14.6323x over previous
import functools

import jax
import jax.numpy as jnp
from jax import lax
from jax.experimental import pallas as pl
from jax.experimental.pallas import tpu as pltpu
from jax.experimental.pallas import tpu_sc as plsc

NC, NS, L = 2, 16, 16
NW = NC * NS
D = 128
BLK = 512
KW = 128


def _round_up(x, m):
    return (x + m - 1) // m * m


def _mesh():
    return plsc.VectorSubcoreMesh(
        core_axis_name="c", subcore_axis_name="s", num_cores=NC, num_subcores=NS
    )


def _make_sc_deg(n_pad, e_pad):
    epw = e_pad // NW
    nwin = epw // KW
    rpt = n_pad // NS

    @functools.partial(
        pl.kernel,
        out_type=jax.ShapeDtypeStruct((NC, n_pad, D), jnp.float32),
        mesh=_mesh(),
        scratch_types=[
            pltpu.VMEM_SHARED((n_pad, D), jnp.float32),
            pltpu.VMEM((KW,), jnp.int32),
            pltpu.VMEM((KW, D), jnp.float32),
        ],
    )
    def deg_kernel(dst_hbm, out_hbm, acc_sh, idx_v, ones_v):
        c = lax.axis_index("c")
        s = lax.axis_index("s")
        wid = s * NC + c

        def fill_z(i, _):
            ones_v[i // (D // L), pl.ds((i % (D // L)) * L, L)] = jnp.zeros((L,), jnp.float32)
            return 0

        lax.fori_loop(0, KW * D // L, fill_z, 0)

        def zcopy(j, _):
            pltpu.sync_copy(ones_v, acc_sh.at[pl.ds(s * rpt + j * KW, KW)])
            return 0

        lax.fori_loop(0, rpt // KW, zcopy, 0)

        def fill_o(i, _):
            ones_v[i // (D // L), pl.ds((i % (D // L)) * L, L)] = jnp.ones((L,), jnp.float32)
            return 0

        lax.fori_loop(0, KW * D // L, fill_o, 0)
        plsc.subcore_barrier()

        def win(w, _):
            base = wid * epw + w * KW
            pltpu.sync_copy(dst_hbm.at[pl.ds(base, KW)], idx_v)
            pltpu.sync_copy(ones_v, acc_sh.at[idx_v], add=True)
            return 0

        lax.fori_loop(0, nwin, win, 0)
        plsc.subcore_barrier()
        pltpu.sync_copy(acc_sh.at[pl.ds(s * rpt, rpt)], out_hbm.at[c, pl.ds(s * rpt, rpt)])

    return deg_kernel


def _make_sc_edge(n_pad, e_pad):
    epw = e_pad // NW
    nwin = epw // KW
    rpt = n_pad // NS

    @functools.partial(
        pl.kernel,
        out_type=jax.ShapeDtypeStruct((NC, n_pad, D), jnp.float32),
        mesh=_mesh(),
        scratch_types=[
            pltpu.VMEM_SHARED((n_pad, D), jnp.float32),
            pltpu.VMEM((KW,), jnp.int32),
            pltpu.VMEM((KW,), jnp.int32),
            pltpu.VMEM((KW, D), jnp.float32),
            pltpu.SemaphoreType.DMA,
        ],
    )
    def edge_kernel(y_hbm, src_hbm, dst_hbm, out_hbm, acc_sh, sidx, didx, rows, sem):
        c = lax.axis_index("c")
        s = lax.axis_index("s")
        wid = s * NC + c

        def fill_z(i, _):
            rows[i // (D // L), pl.ds((i % (D // L)) * L, L)] = jnp.zeros((L,), jnp.float32)
            return 0

        lax.fori_loop(0, KW * D // L, fill_z, 0)

        def zcopy(j, _):
            pltpu.sync_copy(rows, acc_sh.at[pl.ds(s * rpt + j * KW, KW)])
            return 0

        lax.fori_loop(0, rpt // KW, zcopy, 0)
        plsc.subcore_barrier()

        def win(w, _):
            base = wid * epw + w * KW
            pltpu.sync_copy(src_hbm.at[pl.ds(base, KW)], sidx)
            pltpu.sync_copy(dst_hbm.at[pl.ds(base, KW)], didx)
            pltpu.async_copy(y_hbm.at[sidx], rows, sem).wait()
            pltpu.sync_copy(rows, acc_sh.at[didx], add=True)
            return 0

        lax.fori_loop(0, nwin, win, 0)
        plsc.subcore_barrier()
        pltpu.sync_copy(acc_sh.at[pl.ds(s * rpt, rpt)], out_hbm.at[c, pl.ds(s * rpt, rpt)])

    return edge_kernel


def _lane_cumprod(x):
    n_rows = x.shape[0]
    c = x
    sh = 1
    while sh < D:
        c = c * jnp.concatenate(
            [jnp.ones((n_rows, sh), jnp.float32), c[:, : D - sh]], axis=1
        )
        sh *= 2
    return c


def _dense_body(n_real, sphex_ref, wg_ref, wi_ref, bi_ref, xw_ref, intra_ref, cs_ref):
    i = pl.program_id(0)
    sp = sphex_ref[...]
    cosx = jnp.cos(sp)
    sinx = jnp.sin(sp)
    cum = _lane_cumprod(sinx)
    gex = jnp.concatenate([cosx[:, :1], cosx[:, 1:] * cum[:, : D - 1]], axis=1)
    dn = (((1,), (1,)), ((), ()))
    xw_ref[...] = lax.dot_general(gex, wg_ref[...], dn, preferred_element_type=jnp.float32)
    intra_ref[...] = (
        lax.dot_general(gex, wi_ref[...], dn, preferred_element_type=jnp.float32)
        + bi_ref[...]
    )
    rows = i * BLK + lax.broadcasted_iota(jnp.int32, (BLK, D), 0)
    gexm = jnp.where(rows < n_real, gex, 0.0)

    @pl.when(i == 0)
    def _():
        cs_ref[...] = jnp.zeros_like(cs_ref)

    cs_ref[...] += jnp.sum(gexm, axis=0, keepdims=True)


def _scale_body(xw_ref, degp_ref, y_ref):
    dp = degp_ref[...]
    deg = dp[0, :, :1] + dp[1, :, :1]
    dinv = jnp.where(deg > 0.0, lax.rsqrt(jnp.maximum(deg, 1.0)), 0.0)
    y_ref[...] = xw_ref[...] * dinv


def _final_body(n_real, accp_ref, degp_ref, bg_ref, cs_ref, wg_ref, wi_ref, msg_ref, z_ref):
    i = pl.program_id(0)
    a = accp_ref[0] + accp_ref[1]
    dp = degp_ref[...]
    deg = dp[0, :, :1] + dp[1, :, :1]
    dinv = jnp.where(deg > 0.0, lax.rsqrt(jnp.maximum(deg, 1.0)), 0.0)
    msg_ref[...] = a * dinv + bg_ref[...]

    @pl.when(i == 0)
    def _():
        mean = cs_ref[...] * (1.0 / n_real)
        mt = 6.0 * wg_ref[...] + 2.0 * wi_ref[...]
        dn = (((1,), (1,)), ((), ()))
        arow = lax.dot_general(mean, mt, dn, preferred_element_type=jnp.float32)
        g2 = jnp.sum(arow * arow)
        g = jnp.sqrt(g2)
        gs = jnp.maximum(g, 1e-12)
        z_mean = -0.5 * n_real * jnp.sum(mean * arow)
        z_int = n_real * (gs - jnp.log(2.0 * gs) + jnp.log1p(-jnp.exp(-2.0 * gs)))
        z_ref[...] = jnp.broadcast_to(z_mean + z_int, (1, 1))


def kernel(edge_index, batch, sphex, W_g2g, b_g2g, W_intra, b_intra):
    n = sphex.shape[0]
    e = edge_index.shape[1]
    n_pad = _round_up(n, 2048)
    if n_pad == n:
        n_pad += 2048
    e_pad = _round_up(e, NW * KW)
    grid = n_pad // BLK

    pad_e = e_pad - e
    ar = jnp.arange(pad_e, dtype=jnp.int32)
    src_p = jnp.concatenate([edge_index[0], ar % n])
    dst_p = jnp.concatenate([edge_index[1], n + ar % (n_pad - n)])
    sphex_p = jnp.pad(sphex, ((0, n_pad - n), (0, 0)))
    bg2 = b_g2g.reshape(1, D)
    bi2 = b_intra.reshape(1, D)

    sc_deg = _make_sc_deg(n_pad, e_pad)
    sc_edge = _make_sc_edge(n_pad, e_pad)

    degp = sc_deg(dst_p)

    f32 = jnp.float32
    xw, intra, colsum = pl.pallas_call(
        functools.partial(_dense_body, n),
        grid=(grid,),
        in_specs=[
            pl.BlockSpec((BLK, D), lambda i: (i, 0)),
            pl.BlockSpec((D, D), lambda i: (0, 0)),
            pl.BlockSpec((D, D), lambda i: (0, 0)),
            pl.BlockSpec((1, D), lambda i: (0, 0)),
        ],
        out_specs=[
            pl.BlockSpec((BLK, D), lambda i: (i, 0)),
            pl.BlockSpec((BLK, D), lambda i: (i, 0)),
            pl.BlockSpec((1, D), lambda i: (0, 0)),
        ],
        out_shape=[
            jax.ShapeDtypeStruct((n_pad, D), f32),
            jax.ShapeDtypeStruct((n_pad, D), f32),
            jax.ShapeDtypeStruct((1, D), f32),
        ],
    )(sphex_p, W_g2g, W_intra, bi2)

    y = pl.pallas_call(
        _scale_body,
        grid=(grid,),
        in_specs=[
            pl.BlockSpec((BLK, D), lambda i: (i, 0)),
            pl.BlockSpec((NC, BLK, D), lambda i: (0, i, 0)),
        ],
        out_specs=pl.BlockSpec((BLK, D), lambda i: (i, 0)),
        out_shape=jax.ShapeDtypeStruct((n_pad, D), f32),
    )(xw, degp)

    accp = sc_edge(y, src_p, dst_p)

    msg, logz = pl.pallas_call(
        functools.partial(_final_body, n),
        grid=(grid,),
        in_specs=[
            pl.BlockSpec((NC, BLK, D), lambda i: (0, i, 0)),
            pl.BlockSpec((NC, BLK, D), lambda i: (0, i, 0)),
            pl.BlockSpec((1, D), lambda i: (0, 0)),
            pl.BlockSpec((1, D), lambda i: (0, 0)),
            pl.BlockSpec((D, D), lambda i: (0, 0)),
            pl.BlockSpec((D, D), lambda i: (0, 0)),
        ],
        out_specs=[
            pl.BlockSpec((BLK, D), lambda i: (i, 0)),
            pl.BlockSpec((1, 1), lambda i: (0, 0)),
        ],
        out_shape=[
            jax.ShapeDtypeStruct((n_pad, D), f32),
            jax.ShapeDtypeStruct((1, 1), f32),
        ],
    )(accp, degp, bg2, colsum, W_g2g, W_intra)

    return msg[:n], intra[:n], logz


# speedup vs baseline: 23.2344x; 1.5879x over previous
import functools

import jax
import jax.numpy as jnp
from jax import lax
from jax.experimental import pallas as pl
from jax.experimental.pallas import tpu as pltpu
from jax.experimental.pallas import tpu_sc as plsc

NC, NS, L = 2, 16, 16
NW = NC * NS
D = 128
BLK = 512
KW = 128


def _round_up(x, m):
    return (x + m - 1) // m * m


def _mesh():
    return plsc.VectorSubcoreMesh(
        core_axis_name="c", subcore_axis_name="s", num_cores=NC, num_subcores=NS
    )


CH = 8


def _make_sc_deg(n_pad, e_pad):
    epw = e_pad // NW
    nwin = epw // KW
    nch = nwin // CH
    rpt = n_pad // NS

    @functools.partial(
        pl.kernel,
        out_type=jax.ShapeDtypeStruct((NC, n_pad, D), jnp.float32),
        mesh=_mesh(),
        scratch_types=[
            pltpu.VMEM_SHARED((n_pad, D), jnp.float32),
            pltpu.VMEM((nch, CH, KW), jnp.int32),
            pltpu.VMEM((KW, D), jnp.float32),
            pltpu.SemaphoreType.DMA,
        ],
    )
    def deg_kernel(dst_hbm, out_hbm, acc_sh, didx, ones_v, sem):
        c = lax.axis_index("c")
        s = lax.axis_index("s")
        wid = s * NC + c
        pltpu.sync_copy(dst_hbm.at[wid], didx)

        def fill_z(i, _):
            ones_v[i // (D // L), pl.ds((i % (D // L)) * L, L)] = jnp.zeros((L,), jnp.float32)
            return 0

        lax.fori_loop(0, KW * D // L, fill_z, 0)

        def zcopy(j, _):
            pltpu.sync_copy(ones_v, acc_sh.at[pl.ds(s * rpt + j * KW, KW)])
            return 0

        lax.fori_loop(0, rpt // KW, zcopy, 0)

        def fill_o(i, _):
            ones_v[i // (D // L), pl.ds((i % (D // L)) * L, L)] = jnp.ones((L,), jnp.float32)
            return 0

        lax.fori_loop(0, KW * D // L, fill_o, 0)
        plsc.subcore_barrier()

        def group(k, _):
            for j in range(CH):
                pltpu.async_copy(ones_v, acc_sh.at[didx.at[k, j]], sem, add=True)
            for j in range(CH):
                pltpu.make_async_copy(ones_v, acc_sh.at[didx.at[k, j]], sem).wait()
            return 0

        lax.fori_loop(0, nch, group, 0)
        plsc.subcore_barrier()
        pltpu.sync_copy(acc_sh.at[pl.ds(s * rpt, rpt)], out_hbm.at[c, pl.ds(s * rpt, rpt)])

    return deg_kernel


def _make_sc_edge(n_pad, e_pad):
    epw = e_pad // NW
    nwin = epw // KW
    nch = nwin // CH
    rpt = n_pad // NS
    assert nch % 2 == 0

    @functools.partial(
        pl.kernel,
        out_type=jax.ShapeDtypeStruct((NC, n_pad, D), jnp.float32),
        mesh=_mesh(),
        scratch_types=[
            pltpu.VMEM_SHARED((n_pad, D), jnp.float32),
            pltpu.VMEM((CH, KW), jnp.int32),
            pltpu.VMEM((CH, KW), jnp.int32),
            pltpu.VMEM((CH, KW), jnp.int32),
            pltpu.VMEM((CH, KW), jnp.int32),
            pltpu.VMEM((KW, D), jnp.float32),
            pltpu.VMEM((KW, D), jnp.float32),
            pltpu.SemaphoreType.DMA,
            pltpu.SemaphoreType.DMA,
            pltpu.SemaphoreType.DMA,
            pltpu.SemaphoreType.DMA,
        ],
    )
    def edge_kernel(y_hbm, src_hbm, dst_hbm, out_hbm, acc_sh,
                    sidx_a, didx_a, sidx_b, didx_b, rows0, rows1,
                    sem_ia, sem_ib, sem_g0, sem_g1):
        c = lax.axis_index("c")
        s = lax.axis_index("s")
        wid = s * NC + c

        def fill_z(i, _):
            rows0[i // (D // L), pl.ds((i % (D // L)) * L, L)] = jnp.zeros((L,), jnp.float32)
            return 0

        lax.fori_loop(0, KW * D // L, fill_z, 0)

        def zcopy(j, _):
            pltpu.sync_copy(rows0, acc_sh.at[pl.ds(s * rpt + j * KW, KW)])
            return 0

        lax.fori_loop(0, rpt // KW, zcopy, 0)
        plsc.subcore_barrier()

        pltpu.async_copy(src_hbm.at[wid, 0], sidx_a, sem_ia)
        pltpu.async_copy(dst_hbm.at[wid, 0], didx_a, sem_ia)

        def do_chunk(k, sidx, didx, sem_i, sidx_n, didx_n, sem_i_n):
            pltpu.make_async_copy(src_hbm.at[wid, k], sidx, sem_i).wait()
            pltpu.make_async_copy(dst_hbm.at[wid, k], didx, sem_i).wait()

            @pl.when(k + 1 < nch)
            def _():
                pltpu.async_copy(src_hbm.at[wid, k + 1], sidx_n, sem_i_n)
                pltpu.async_copy(dst_hbm.at[wid, k + 1], didx_n, sem_i_n)

            pltpu.async_copy(y_hbm.at[sidx.at[0]], rows0, sem_g0)
            for j in range(CH):
                rj, sj = (rows0, sem_g0) if j % 2 == 0 else (rows1, sem_g1)
                if j + 1 < CH:
                    rn, sn = (rows1, sem_g1) if j % 2 == 0 else (rows0, sem_g0)
                    pltpu.async_copy(y_hbm.at[sidx.at[j + 1]], rn, sn)
                pltpu.make_async_copy(y_hbm.at[sidx.at[j]], rj, sj).wait()
                pltpu.sync_copy(rj, acc_sh.at[didx.at[j]], add=True)

        def pair(p, _):
            do_chunk(2 * p, sidx_a, didx_a, sem_ia, sidx_b, didx_b, sem_ib)
            do_chunk(2 * p + 1, sidx_b, didx_b, sem_ib, sidx_a, didx_a, sem_ia)
            return 0

        lax.fori_loop(0, nch // 2, pair, 0)
        plsc.subcore_barrier()
        pltpu.sync_copy(acc_sh.at[pl.ds(s * rpt, rpt)], out_hbm.at[c, pl.ds(s * rpt, rpt)])

    return edge_kernel


def _lane_cumprod(x):
    n_rows = x.shape[0]
    c = x
    sh = 1
    while sh < D:
        c = c * jnp.concatenate(
            [jnp.ones((n_rows, sh), jnp.float32), c[:, : D - sh]], axis=1
        )
        sh *= 2
    return c


def _dense_body(n_real, sphex_ref, wg_ref, wi_ref, bi_ref, xw_ref, intra_ref, cs_ref):
    i = pl.program_id(0)
    sp = sphex_ref[...]
    cosx = jnp.cos(sp)
    sinx = jnp.sin(sp)
    cum = _lane_cumprod(sinx)
    gex = jnp.concatenate([cosx[:, :1], cosx[:, 1:] * cum[:, : D - 1]], axis=1)
    dn = (((1,), (1,)), ((), ()))
    xw_ref[...] = lax.dot_general(gex, wg_ref[...], dn, preferred_element_type=jnp.float32)
    intra_ref[...] = (
        lax.dot_general(gex, wi_ref[...], dn, preferred_element_type=jnp.float32)
        + bi_ref[...]
    )
    rows = i * BLK + lax.broadcasted_iota(jnp.int32, (BLK, D), 0)
    gexm = jnp.where(rows < n_real, gex, 0.0)

    @pl.when(i == 0)
    def _():
        cs_ref[...] = jnp.zeros_like(cs_ref)

    cs_ref[...] += jnp.sum(gexm, axis=0, keepdims=True)


def _scale_body(xw_ref, degp_ref, y_ref, dinv_ref):
    dp = degp_ref[...]
    deg = dp[0, :, :1] + dp[1, :, :1]
    dinv = jnp.where(deg > 0.0, lax.rsqrt(jnp.maximum(deg, 1.0)), 0.0)
    y_ref[...] = xw_ref[...] * dinv
    dinv_ref[...] = dinv


def _final_body(n_real, accp_ref, dinv_ref, bg_ref, cs_ref, wg_ref, wi_ref, msg_ref, z_ref):
    i = pl.program_id(0)
    a = accp_ref[0] + accp_ref[1]
    msg_ref[...] = a * dinv_ref[...] + bg_ref[...]

    @pl.when(i == 0)
    def _():
        mean = cs_ref[...] * (1.0 / n_real)
        mt = 6.0 * wg_ref[...] + 2.0 * wi_ref[...]
        dn = (((1,), (1,)), ((), ()))
        arow = lax.dot_general(mean, mt, dn, preferred_element_type=jnp.float32)
        g2 = jnp.sum(arow * arow)
        g = jnp.sqrt(g2)
        gs = jnp.maximum(g, 1e-12)
        z_mean = -0.5 * n_real * jnp.sum(mean * arow)
        z_int = n_real * (gs - jnp.log(2.0 * gs) + jnp.log1p(-jnp.exp(-2.0 * gs)))
        z_ref[...] = jnp.broadcast_to(z_mean + z_int, (1, 1))


def kernel(edge_index, batch, sphex, W_g2g, b_g2g, W_intra, b_intra):
    n = sphex.shape[0]
    e = edge_index.shape[1]
    n_pad = _round_up(n, 2048)
    if n_pad == n:
        n_pad += 2048
    e_pad = _round_up(e, 2 * NW * KW * CH)
    grid = n_pad // BLK
    epw = e_pad // NW
    nwin = epw // KW
    nch = nwin // CH

    pad_e = e_pad - e
    ar = jnp.arange(pad_e, dtype=jnp.int32)
    src_p = jnp.concatenate([edge_index[0], ar % n]).reshape(NW, nch, CH, KW)
    dst_p = jnp.concatenate([edge_index[1], n + ar % (n_pad - n)]).reshape(NW, nch, CH, KW)
    sphex_p = jnp.pad(sphex, ((0, n_pad - n), (0, 0)))
    bg2 = b_g2g.reshape(1, D)
    bi2 = b_intra.reshape(1, D)

    sc_deg = _make_sc_deg(n_pad, e_pad)
    sc_edge = _make_sc_edge(n_pad, e_pad)

    degp = sc_deg(dst_p)

    f32 = jnp.float32
    xw, intra, colsum = pl.pallas_call(
        functools.partial(_dense_body, n),
        grid=(grid,),
        in_specs=[
            pl.BlockSpec((BLK, D), lambda i: (i, 0)),
            pl.BlockSpec((D, D), lambda i: (0, 0)),
            pl.BlockSpec((D, D), lambda i: (0, 0)),
            pl.BlockSpec((1, D), lambda i: (0, 0)),
        ],
        out_specs=[
            pl.BlockSpec((BLK, D), lambda i: (i, 0)),
            pl.BlockSpec((BLK, D), lambda i: (i, 0)),
            pl.BlockSpec((1, D), lambda i: (0, 0)),
        ],
        out_shape=[
            jax.ShapeDtypeStruct((n_pad, D), f32),
            jax.ShapeDtypeStruct((n_pad, D), f32),
            jax.ShapeDtypeStruct((1, D), f32),
        ],
    )(sphex_p, W_g2g, W_intra, bi2)

    y, dinv = pl.pallas_call(
        _scale_body,
        grid=(grid,),
        in_specs=[
            pl.BlockSpec((BLK, D), lambda i: (i, 0)),
            pl.BlockSpec((NC, BLK, D), lambda i: (0, i, 0)),
        ],
        out_specs=[
            pl.BlockSpec((BLK, D), lambda i: (i, 0)),
            pl.BlockSpec((BLK, 1), lambda i: (i, 0)),
        ],
        out_shape=[
            jax.ShapeDtypeStruct((n_pad, D), f32),
            jax.ShapeDtypeStruct((n_pad, 1), f32),
        ],
    )(xw, degp)

    accp = sc_edge(y, src_p, dst_p)

    msg, logz = pl.pallas_call(
        functools.partial(_final_body, n),
        grid=(grid,),
        in_specs=[
            pl.BlockSpec((NC, BLK, D), lambda i: (0, i, 0)),
            pl.BlockSpec((BLK, 1), lambda i: (i, 0)),
            pl.BlockSpec((1, D), lambda i: (0, 0)),
            pl.BlockSpec((1, D), lambda i: (0, 0)),
            pl.BlockSpec((D, D), lambda i: (0, 0)),
            pl.BlockSpec((D, D), lambda i: (0, 0)),
        ],
        out_specs=[
            pl.BlockSpec((BLK, D), lambda i: (i, 0)),
            pl.BlockSpec((1, 1), lambda i: (0, 0)),
        ],
        out_shape=[
            jax.ShapeDtypeStruct((n_pad, D), f32),
            jax.ShapeDtypeStruct((1, 1), f32),
        ],
    )(accp, dinv, bg2, colsum, W_g2g, W_intra)

    return msg[:n], intra[:n], logz
